# baseline (speedup 1.0000x reference)
import functools
import jax
import jax.numpy as jnp
from jax import lax
from jax.experimental import pallas as pl
from jax.experimental.pallas import tpu as pltpu
from jax.experimental.pallas import tpu_sc as plsc

N_NODES = 100000
N_GRAPHS = 1024
F_X = 4
F_U = 1
HID = F_U + F_X
ROW = 8
CHUNK = 4000
NW = N_NODES // CHUNK
NODES_PER_128 = 128 // ROW
XROWS = N_NODES // NODES_PER_128
WROWS = CHUNK // NODES_PER_128

_mesh = plsc.VectorSubcoreMesh(core_axis_name="c", subcore_axis_name="s")


def _sc_pool_body(xp_hbm, b_hbm, zero_hbm, acc_hbm, x_v, braw_v, idx_v,
                  acc_sh, sem):
    c = lax.axis_index("c")
    s = lax.axis_index("s")
    wid = s * 2 + c

    @pl.when(s == 0)
    def _():
        pltpu.sync_copy(zero_hbm, acc_sh)

    plsc.subcore_barrier()

    @pl.when(wid < NW)
    def _():
        r0 = wid * WROWS
        copies = [
            pltpu.async_copy(
                xp_hbm.at[pl.ds(r0, WROWS), pl.ds(ROW * cb, ROW)],
                x_v.at[pl.ds(WROWS * cb, WROWS), :],
                sem,
            )
            for cb in range(NODES_PER_128)
        ]
        pltpu.sync_copy(b_hbm.at[pl.ds(wid * CHUNK, CHUNK)], braw_v)
        lanes = lax.iota(jnp.int32, 16) * NODES_PER_128
        for cb in range(NODES_PER_128):
            for rb in range(16):
                r = min(16 * rb, WROWS - 16)
                vec = plsc.load_gather(
                    braw_v, [lanes + (NODES_PER_128 * r + cb)])
                idx_v[pl.ds(WROWS * cb + r, 16)] = vec
        for cp in copies:
            cp.wait()
        pltpu.sync_copy(x_v, acc_sh.at[idx_v], add=True)

    plsc.subcore_barrier()

    @pl.when(s == 0)
    def _():
        pltpu.sync_copy(acc_sh, acc_hbm.at[c])


_sc_pool = functools.partial(
    pl.kernel,
    out_type=jax.ShapeDtypeStruct((2, N_GRAPHS, ROW), jnp.float32),
    mesh=_mesh,
    compiler_params=pltpu.CompilerParams(use_tc_tiling_on_sc=False,
                                         needs_layout_passes=False),
    scratch_types=[
        pltpu.VMEM((CHUNK, ROW), jnp.float32),
        pltpu.VMEM((CHUNK,), jnp.int32),
        pltpu.VMEM((CHUNK,), jnp.int32),
        pltpu.VMEM_SHARED((N_GRAPHS, ROW), jnp.float32),
        pltpu.SemaphoreType.DMA,
    ],
)(_sc_pool_body)


def _mlp_body(p_ref, u_ref, w1t_ref, b1_ref, w2t_ref, b2_ref, o_ref):
    tot = p_ref[0] + p_ref[1]
    pooled = tot[:, :F_X] / jnp.maximum(tot[:, F_X:F_X + 1], 1.0)
    feats = jnp.concatenate([u_ref[...], pooled], axis=1)
    h = jax.lax.dot(feats, w1t_ref[...],
                    precision=jax.lax.Precision.HIGHEST) + b1_ref[...]
    h = jnp.where(h > 0, h, 0.1 * h)
    o_ref[...] = jax.lax.dot(h, w2t_ref[...],
                             precision=jax.lax.Precision.HIGHEST) + b2_ref[...]


def kernel(x, edge_index, edge_attr, u, batch, W1, b1, W2, b2):
    del edge_index, edge_attr
    xp = jnp.concatenate(
        [x, jnp.ones((N_NODES, 1), jnp.float32),
         jnp.zeros((N_NODES, ROW - F_X - 1), jnp.float32)], axis=1)
    xp128 = xp.reshape(XROWS, 128)
    b32 = batch.astype(jnp.int32)
    zero = jnp.zeros((N_GRAPHS, ROW), jnp.float32)

    acc = _sc_pool(xp128, b32, zero)

    y = pl.pallas_call(
        _mlp_body,
        out_shape=jax.ShapeDtypeStruct((N_GRAPHS, F_U), jnp.float32),
    )(
        acc,
        u,
        W1.T,
        b1.reshape(1, HID),
        W2.T,
        b2.reshape(1, F_U),
    )
    return y


# speedup vs baseline: 2.1553x; 2.1553x over previous
import functools
import jax
import jax.numpy as jnp
from jax import lax
from jax.experimental import pallas as pl
from jax.experimental.pallas import tpu as pltpu
from jax.experimental.pallas import tpu_sc as plsc

N_NODES = 100000
N_GRAPHS = 1024
F_X = 4
F_U = 1
HID = F_U + F_X
ROW = 8
CHUNK = 4000
NW = N_NODES // CHUNK
NVEC = CHUNK // 16

_mesh = plsc.VectorSubcoreMesh(core_axis_name="c", subcore_axis_name="s")


def _sc_pool_body(x0_hbm, x1_hbm, x2_hbm, x3_hbm, b_hbm, zero_hbm, acc_hbm,
                  xcol_v, x_v, idx_v, acc_sh, sem):
    c = lax.axis_index("c")
    s = lax.axis_index("s")
    wid = s * 2 + c

    @pl.when(s == 0)
    def _():
        pltpu.sync_copy(zero_hbm, acc_sh)

    plsc.subcore_barrier()

    @pl.when(wid < NW)
    def _():
        n0 = wid * CHUNK
        copies = [
            pltpu.async_copy(xf.at[pl.ds(n0, CHUNK)], xcol_v.at[f, :], sem)
            for f, xf in enumerate((x0_hbm, x1_hbm, x2_hbm, x3_hbm))
        ]
        pltpu.sync_copy(b_hbm.at[pl.ds(n0, CHUNK)], idx_v)
        for cp in copies:
            cp.wait()

        lanes = lax.iota(jnp.int32, 16)
        ones_vec = jnp.full((16,), 1.0, jnp.float32)
        col_ones = jnp.full((16,), F_X, jnp.int32)

        def build(k, carry):
            rows = lanes + 16 * k
            for f in range(F_X):
                vals = xcol_v[f, pl.ds(16 * k, 16)]
                plsc.store_scatter(x_v, [rows, jnp.full((16,), f, jnp.int32)],
                                   vals)
            plsc.store_scatter(x_v, [rows, col_ones], ones_vec)
            return carry

        lax.fori_loop(0, NVEC, build, 0)
        pltpu.sync_copy(x_v, acc_sh.at[idx_v], add=True)

    plsc.subcore_barrier()

    @pl.when(s == 0)
    def _():
        pltpu.sync_copy(acc_sh, acc_hbm.at[c])


_sc_pool = functools.partial(
    pl.kernel,
    out_type=jax.ShapeDtypeStruct((2, N_GRAPHS, ROW), jnp.float32),
    mesh=_mesh,
    compiler_params=pltpu.CompilerParams(use_tc_tiling_on_sc=False,
                                         needs_layout_passes=False),
    scratch_types=[
        pltpu.VMEM((F_X, CHUNK), jnp.float32),
        pltpu.VMEM((CHUNK, ROW), jnp.float32),
        pltpu.VMEM((CHUNK,), jnp.int32),
        pltpu.VMEM_SHARED((N_GRAPHS, ROW), jnp.float32),
        pltpu.SemaphoreType.DMA,
    ],
)(_sc_pool_body)


def _mlp_body(p_ref, u_ref, w1t_ref, b1_ref, w2t_ref, b2_ref, o_ref):
    tot = p_ref[0] + p_ref[1]
    pooled = tot[:, :F_X] / jnp.maximum(tot[:, F_X:F_X + 1], 1.0)
    feats = jnp.concatenate([u_ref[...], pooled], axis=1)
    h = jax.lax.dot(feats, w1t_ref[...],
                    precision=jax.lax.Precision.HIGHEST) + b1_ref[...]
    h = jnp.where(h > 0, h, 0.1 * h)
    o_ref[...] = jax.lax.dot(h, w2t_ref[...],
                             precision=jax.lax.Precision.HIGHEST) + b2_ref[...]


def kernel(x, edge_index, edge_attr, u, batch, W1, b1, W2, b2):
    del edge_index, edge_attr
    b32 = batch.astype(jnp.int32)
    zero = jnp.zeros((N_GRAPHS, ROW), jnp.float32)

    acc = _sc_pool(x[:, 0], x[:, 1], x[:, 2], x[:, 3], b32, zero)

    y = pl.pallas_call(
        _mlp_body,
        out_shape=jax.ShapeDtypeStruct((N_GRAPHS, F_U), jnp.float32),
    )(
        acc,
        u,
        W1.T,
        b1.reshape(1, HID),
        W2.T,
        b2.reshape(1, F_U),
    )
    return y


# speedup vs baseline: 2.4520x; 1.1376x over previous
import functools
import jax
import jax.numpy as jnp
from jax import lax
from jax.experimental import pallas as pl
from jax.experimental.pallas import tpu as pltpu
from jax.experimental.pallas import tpu_sc as plsc

N_NODES = 100000
N_GRAPHS = 1024
F_X = 4
F_U = 1
HID = F_U + F_X
ROW = 8
CHUNK = 4000
NW = N_NODES // CHUNK
NVEC = CHUNK // 16

_mesh = plsc.VectorSubcoreMesh(core_axis_name="c", subcore_axis_name="s")


def _sc_pool_body(xt_hbm, b_hbm, zero_hbm, acc_hbm,
                  xcol_v, x_v, idx_v, acc_sh, sem):
    c = lax.axis_index("c")
    s = lax.axis_index("s")
    wid = s * 2 + c

    @pl.when(s == 0)
    def _():
        pltpu.sync_copy(zero_hbm, acc_sh)

    plsc.subcore_barrier()

    @pl.when(wid < NW)
    def _():
        n0 = wid * CHUNK
        copies = [
            pltpu.async_copy(xt_hbm.at[pl.ds(f * N_NODES + n0, CHUNK)],
                             xcol_v.at[f, :], sem)
            for f in range(F_X)
        ]
        pltpu.sync_copy(b_hbm.at[pl.ds(n0, CHUNK)], idx_v)
        for cp in copies:
            cp.wait()

        lanes = lax.iota(jnp.int32, 16)
        ones_vec = jnp.full((16,), 1.0, jnp.float32)
        col_ones = jnp.full((16,), F_X, jnp.int32)

        def build(k, carry):
            rows = lanes + 16 * k
            for f in range(F_X):
                vals = xcol_v[f, pl.ds(16 * k, 16)]
                plsc.store_scatter(x_v, [rows, jnp.full((16,), f, jnp.int32)],
                                   vals)
            plsc.store_scatter(x_v, [rows, col_ones], ones_vec)
            return carry

        lax.fori_loop(0, NVEC, build, 0)
        pltpu.sync_copy(x_v, acc_sh.at[idx_v], add=True)

    plsc.subcore_barrier()

    @pl.when(s == 0)
    def _():
        pltpu.sync_copy(acc_sh, acc_hbm.at[c])


_sc_pool = functools.partial(
    pl.kernel,
    out_type=jax.ShapeDtypeStruct((2, N_GRAPHS, ROW), jnp.float32),
    mesh=_mesh,
    compiler_params=pltpu.CompilerParams(use_tc_tiling_on_sc=False,
                                         needs_layout_passes=False),
    scratch_types=[
        pltpu.VMEM((F_X, CHUNK), jnp.float32),
        pltpu.VMEM((CHUNK, ROW), jnp.float32),
        pltpu.VMEM((CHUNK,), jnp.int32),
        pltpu.VMEM_SHARED((N_GRAPHS, ROW), jnp.float32),
        pltpu.SemaphoreType.DMA,
    ],
)(_sc_pool_body)


def _mlp_body(p_ref, u_ref, w1t_ref, b1_ref, w2t_ref, b2_ref, o_ref):
    tot = p_ref[0] + p_ref[1]
    pooled = tot[:, :F_X] / jnp.maximum(tot[:, F_X:F_X + 1], 1.0)
    feats = jnp.concatenate([u_ref[...], pooled], axis=1)
    h = jax.lax.dot(feats, w1t_ref[...],
                    precision=jax.lax.Precision.HIGHEST) + b1_ref[...]
    h = jnp.where(h > 0, h, 0.1 * h)
    o_ref[...] = jax.lax.dot(h, w2t_ref[...],
                             precision=jax.lax.Precision.HIGHEST) + b2_ref[...]


def kernel(x, edge_index, edge_attr, u, batch, W1, b1, W2, b2):
    del edge_index, edge_attr
    b32 = batch.astype(jnp.int32)
    zero = jnp.zeros((N_GRAPHS, ROW), jnp.float32)

    acc = _sc_pool(x.T.reshape(F_X * N_NODES), b32, zero)

    y = pl.pallas_call(
        _mlp_body,
        out_shape=jax.ShapeDtypeStruct((N_GRAPHS, F_U), jnp.float32),
    )(
        acc,
        u,
        W1.T,
        b1.reshape(1, HID),
        W2.T,
        b2.reshape(1, F_U),
    )
    return y
